# baseline (speedup 1.0000x reference)
import jax
import jax.numpy as jnp
from jax import lax
from jax.experimental import pallas as pl
from jax.experimental.pallas import tpu as pltpu
from jax.experimental.pallas import tpu_sc as plsc

_NC = 2
_NS = 16
_NW = _NC * _NS
_C = 80
_NBUF = 5
_LAG = 2


def _gather_body(x_hbm, idx_hbm, out_hbm, idx_v, rows_v, gsem, ssem):
    bpw = idx_hbm.shape[0] // _NW
    nchunks = bpw // _C
    ngroups = nchunks // _NBUF
    wid = lax.axis_index("s") * _NC + lax.axis_index("c")
    base0 = wid * bpw

    pltpu.sync_copy(idx_hbm.at[pl.ds(base0, bpw)], idx_v)

    def gather(j, b):
        return pltpu.make_async_copy(
            x_hbm.at[idx_v.at[pl.ds(j * _C, _C)]], rows_v.at[b], gsem.at[b])

    def store(j, b):
        return pltpu.make_async_copy(
            rows_v.at[b], out_hbm.at[pl.ds(base0 + j * _C, _C)], ssem.at[b])

    for t in range(_NBUF):
        gather(t, t).start()
        if t >= _LAG:
            gather(t - _LAG, t - _LAG).wait()
            store(t - _LAG, t - _LAG).start()

    def body(g, carry):
        for b in range(_NBUF):
            j = g * _NBUF + b
            store(j - _NBUF, b).wait()
            gather(j, b).start()
            b2 = (b - _LAG) % _NBUF
            gather(j - _LAG, b2).wait()
            store(j - _LAG, b2).start()
        return carry

    lax.fori_loop(1, ngroups, body, 0)

    for j in range(nchunks - _LAG, nchunks):
        gather(j, j % _NBUF).wait()
        store(j, j % _NBUF).start()
    for j in range(nchunks - _NBUF, nchunks):
        store(j, j % _NBUF).wait()


def kernel(x, seed_inverse_ids):
    idx = seed_inverse_ids.astype(jnp.int32)
    B = idx.shape[0]
    mesh = plsc.VectorSubcoreMesh(core_axis_name="c", subcore_axis_name="s")
    k = pl.kernel(
        _gather_body,
        mesh=mesh,
        out_type=jax.ShapeDtypeStruct((B, x.shape[1]), x.dtype),
        scratch_types=[
            pltpu.VMEM((B // _NW,), jnp.int32),
            pltpu.VMEM((_NBUF, _C, x.shape[1]), jnp.float32),
            pltpu.SemaphoreType.DMA((_NBUF,)),
            pltpu.SemaphoreType.DMA((_NBUF,)),
        ],
    )
    return k(x, idx)


# speedup vs baseline: 1.5641x; 1.5641x over previous
import jax
import jax.numpy as jnp
from jax import lax
from jax.experimental import pallas as pl
from jax.experimental.pallas import tpu as pltpu
from jax.experimental.pallas import tpu_sc as plsc

_NC = 2
_NS = 16
_NW = _NC * _NS
_C = 40
_NBUF = 5
_LAG = 2


def _gather_body(x_hbm, idx_hbm, out_hbm, x_sp, idx_v, rows_v, gsem, ssem):
    bpw = idx_hbm.shape[0] // _NW
    nchunks = bpw // _C
    ngroups = nchunks // _NBUF
    sid = lax.axis_index("s")
    wid = sid * _NC + lax.axis_index("c")
    base0 = wid * bpw

    nrows = x_hbm.shape[0]
    rows_per_tile = (nrows // _NS) // 8 * 8
    rem = nrows - rows_per_tile * _NS
    pltpu.sync_copy(x_hbm.at[pl.ds(sid * rows_per_tile, rows_per_tile)],
                    x_sp.at[pl.ds(sid * rows_per_tile, rows_per_tile)])
    if rem:
        @pl.when(sid == 0)
        def _():
            pltpu.sync_copy(x_hbm.at[pl.ds(rows_per_tile * _NS, rem)],
                            x_sp.at[pl.ds(rows_per_tile * _NS, rem)])

    pltpu.sync_copy(idx_hbm.at[pl.ds(base0, bpw)], idx_v)
    plsc.subcore_barrier()

    def gather(j, b):
        return pltpu.make_async_copy(
            x_sp.at[idx_v.at[pl.ds(j * _C, _C)]], rows_v.at[b], gsem.at[b])

    def store(j, b):
        return pltpu.make_async_copy(
            rows_v.at[b], out_hbm.at[pl.ds(base0 + j * _C, _C)], ssem.at[b])

    for t in range(_NBUF):
        gather(t, t).start()
        if t >= _LAG:
            gather(t - _LAG, t - _LAG).wait()
            store(t - _LAG, t - _LAG).start()

    def body(g, carry):
        for b in range(_NBUF):
            j = g * _NBUF + b
            store(j - _NBUF, b).wait()
            gather(j, b).start()
            b2 = (b - _LAG) % _NBUF
            gather(j - _LAG, b2).wait()
            store(j - _LAG, b2).start()
        return carry

    lax.fori_loop(1, ngroups, body, 0)

    for j in range(nchunks - _LAG, nchunks):
        gather(j, j % _NBUF).wait()
        store(j, j % _NBUF).start()
    for j in range(nchunks - _NBUF, nchunks):
        store(j, j % _NBUF).wait()


def kernel(x, seed_inverse_ids):
    idx = seed_inverse_ids.astype(jnp.int32)
    B = idx.shape[0]
    mesh = plsc.VectorSubcoreMesh(core_axis_name="c", subcore_axis_name="s")
    k = pl.kernel(
        _gather_body,
        mesh=mesh,
        out_type=jax.ShapeDtypeStruct((B, x.shape[1]), x.dtype),
        scratch_types=[
            pltpu.VMEM_SHARED(x.shape, x.dtype),
            pltpu.VMEM((B // _NW,), jnp.int32),
            pltpu.VMEM((_NBUF, _C, x.shape[1]), jnp.float32),
            pltpu.SemaphoreType.DMA((_NBUF,)),
            pltpu.SemaphoreType.DMA((_NBUF,)),
        ],
    )
    return k(x, idx)


# speedup vs baseline: 1.5676x; 1.0023x over previous
import jax
import jax.numpy as jnp
from jax import lax
from jax.experimental import pallas as pl
from jax.experimental.pallas import tpu as pltpu
from jax.experimental.pallas import tpu_sc as plsc

_NC = 2
_NS = 16
_NW = _NC * _NS
_C = 80
_NBUF = 4
_LAG = 2


def _gather_body(x_hbm, idx_hbm, out_hbm, x_sp, idx_v, rows_v,
                 stage_sem, gsem, ssem):
    bpw = idx_hbm.shape[0] // _NW
    nchunks = bpw // _C
    nfull = (nchunks - 1) // _NBUF * _NBUF
    sid = lax.axis_index("s")
    wid = sid * _NC + lax.axis_index("c")
    base0 = wid * bpw

    nrows = x_hbm.shape[0]
    rows_per_tile = (nrows // _NS) // 8 * 8
    rem = nrows - rows_per_tile * _NS
    stage = pltpu.make_async_copy(
        x_hbm.at[pl.ds(sid * rows_per_tile, rows_per_tile)],
        x_sp.at[pl.ds(sid * rows_per_tile, rows_per_tile)], stage_sem)
    stage.start()
    pltpu.sync_copy(idx_hbm.at[pl.ds(base0, bpw)], idx_v)
    stage.wait()
    if rem:
        @pl.when(sid == 0)
        def _():
            pltpu.sync_copy(x_hbm.at[pl.ds(rows_per_tile * _NS, rem)],
                            x_sp.at[pl.ds(rows_per_tile * _NS, rem)])
    plsc.subcore_barrier()

    def gather(j, b):
        return pltpu.make_async_copy(
            x_sp.at[idx_v.at[pl.ds(j * _C, _C)]], rows_v.at[b], gsem.at[b])

    def store(j, b):
        return pltpu.make_async_copy(
            rows_v.at[b], out_hbm.at[pl.ds(base0 + j * _C, _C)], ssem.at[b])

    for t in range(_NBUF):
        gather(t, t).start()
        if t >= _LAG:
            gather(t - _LAG, t - _LAG).wait()
            store(t - _LAG, t - _LAG).start()

    def body(g, carry):
        for b in range(_NBUF):
            j = g * _NBUF + b
            store(j - _NBUF, b).wait()
            gather(j, b).start()
            b2 = (b - _LAG) % _NBUF
            gather(j - _LAG, b2).wait()
            store(j - _LAG, b2).start()
        return carry

    lax.fori_loop(1, nfull // _NBUF, body, 0)

    for j in range(nfull - _LAG, nfull):
        gather(j, j % _NBUF).wait()
        store(j, j % _NBUF).start()
    for j in range(nfull - _NBUF, nfull):
        store(j, j % _NBUF).wait()
    for j in range(nfull, nchunks):
        gather(j, 0).start()
        gather(j, 0).wait()
        store(j, 0).start()
        store(j, 0).wait()


def kernel(x, seed_inverse_ids):
    idx = seed_inverse_ids.astype(jnp.int32)
    B = idx.shape[0]
    mesh = plsc.VectorSubcoreMesh(core_axis_name="c", subcore_axis_name="s")
    k = pl.kernel(
        _gather_body,
        mesh=mesh,
        out_type=jax.ShapeDtypeStruct((B, x.shape[1]), x.dtype),
        scratch_types=[
            pltpu.VMEM_SHARED(x.shape, x.dtype),
            pltpu.VMEM((B // _NW,), jnp.int32),
            pltpu.VMEM((_NBUF, _C, x.shape[1]), jnp.float32),
            pltpu.SemaphoreType.DMA,
            pltpu.SemaphoreType.DMA((_NBUF,)),
            pltpu.SemaphoreType.DMA((_NBUF,)),
        ],
    )
    return k(x, idx)


# speedup vs baseline: 1.5850x; 1.0111x over previous
import jax
import jax.numpy as jnp
from jax import lax
from jax.experimental import pallas as pl
from jax.experimental.pallas import tpu as pltpu
from jax.experimental.pallas import tpu_sc as plsc

_NC = 2
_NS = 16
_NW = _NC * _NS
_C = 40
_NBUF = 5
_LAG = 2


def _gather_body(x_hbm, idx_hbm, out_hbm, x_sp, idx_v, rows_v,
                 stage_sem, gsem, ssem):
    bpw = idx_hbm.shape[0] // _NW
    nchunks = bpw // _C
    nfull = nchunks // _NBUF * _NBUF
    sid = lax.axis_index("s")
    wid = sid * _NC + lax.axis_index("c")
    base0 = wid * bpw

    nrows = x_hbm.shape[0]
    rows_per_tile = (nrows // _NS) // 8 * 8
    rem = nrows - rows_per_tile * _NS
    stage = pltpu.make_async_copy(
        x_hbm.at[pl.ds(sid * rows_per_tile, rows_per_tile)],
        x_sp.at[pl.ds(sid * rows_per_tile, rows_per_tile)], stage_sem)
    stage.start()
    pltpu.sync_copy(idx_hbm.at[pl.ds(base0, bpw)], idx_v)
    stage.wait()
    if rem:
        @pl.when(sid == 0)
        def _():
            pltpu.sync_copy(x_hbm.at[pl.ds(rows_per_tile * _NS, rem)],
                            x_sp.at[pl.ds(rows_per_tile * _NS, rem)])
    plsc.subcore_barrier()

    def gather(j, b):
        return pltpu.make_async_copy(
            x_sp.at[idx_v.at[pl.ds(j * _C, _C)]], rows_v.at[b], gsem.at[b])

    def store(j, b):
        return pltpu.make_async_copy(
            rows_v.at[b], out_hbm.at[pl.ds(base0 + j * _C, _C)], ssem.at[b])

    for t in range(_NBUF):
        gather(t, t).start()
        if t >= _LAG:
            gather(t - _LAG, t - _LAG).wait()
            store(t - _LAG, t - _LAG).start()

    def body(g, carry):
        for b in range(_NBUF):
            j = g * _NBUF + b
            store(j - _NBUF, b).wait()
            gather(j, b).start()
            b2 = (b - _LAG) % _NBUF
            gather(j - _LAG, b2).wait()
            store(j - _LAG, b2).start()
        return carry

    lax.fori_loop(1, nfull // _NBUF, body, 0)

    for j in range(nfull - _LAG, nfull):
        gather(j, j % _NBUF).wait()
        store(j, j % _NBUF).start()
    for j in range(nfull - _NBUF, nfull):
        store(j, j % _NBUF).wait()
    for j in range(nfull, nchunks):
        gather(j, 0).start()
        gather(j, 0).wait()
        store(j, 0).start()
        store(j, 0).wait()


def kernel(x, seed_inverse_ids):
    idx = seed_inverse_ids.astype(jnp.int32)
    B = idx.shape[0]
    mesh = plsc.VectorSubcoreMesh(core_axis_name="c", subcore_axis_name="s")
    k = pl.kernel(
        _gather_body,
        mesh=mesh,
        out_type=jax.ShapeDtypeStruct((B, x.shape[1]), x.dtype),
        scratch_types=[
            pltpu.VMEM_SHARED(x.shape, x.dtype),
            pltpu.VMEM((B // _NW,), jnp.int32),
            pltpu.VMEM((_NBUF, _C, x.shape[1]), jnp.float32),
            pltpu.SemaphoreType.DMA,
            pltpu.SemaphoreType.DMA((_NBUF,)),
            pltpu.SemaphoreType.DMA((_NBUF,)),
        ],
    )
    return k(x, idx)


# speedup vs baseline: 1.5924x; 1.0047x over previous
import jax
import jax.numpy as jnp
from jax import lax
from jax.experimental import pallas as pl
from jax.experimental.pallas import tpu as pltpu
from jax.experimental.pallas import tpu_sc as plsc

_NC = 2
_NS = 16
_NW = _NC * _NS
_C = 40
_NBUF = 5
_LAG = 2


def _gather_body(x_hbm, idx_hbm, out_hbm, x_sp, idx_v, rows_v,
                 stage_sem, gsem, ssem):
    bpw = idx_hbm.shape[0] // _NW
    nchunks = bpw // _C
    nfull = nchunks // _NBUF * _NBUF
    sid = lax.axis_index("s")
    wid = sid * _NC + lax.axis_index("c")
    base0 = wid * bpw

    nrows = x_hbm.shape[0]
    rows_per_tile = (nrows // _NS) // 8 * 8
    rem = nrows - rows_per_tile * _NS
    stage = pltpu.make_async_copy(
        x_hbm.at[pl.ds(sid * rows_per_tile, rows_per_tile)],
        x_sp.at[pl.ds(sid * rows_per_tile, rows_per_tile)], stage_sem)
    stage.start()
    pltpu.sync_copy(idx_hbm.at[pl.ds(base0, bpw)], idx_v)

    def gather_hbm(j, b):
        return pltpu.make_async_copy(
            x_hbm.at[idx_v.at[pl.ds(j * _C, _C)]], rows_v.at[b], gsem.at[b])

    def gather(j, b):
        return pltpu.make_async_copy(
            x_sp.at[idx_v.at[pl.ds(j * _C, _C)]], rows_v.at[b], gsem.at[b])

    def store(j, b):
        return pltpu.make_async_copy(
            rows_v.at[b], out_hbm.at[pl.ds(base0 + j * _C, _C)], ssem.at[b])

    for t in range(_NBUF):
        gather_hbm(t, t).start()
        if t >= _LAG:
            gather_hbm(t - _LAG, t - _LAG).wait()
            store(t - _LAG, t - _LAG).start()
    stage.wait()
    if rem:
        @pl.when(sid == 0)
        def _():
            pltpu.sync_copy(x_hbm.at[pl.ds(rows_per_tile * _NS, rem)],
                            x_sp.at[pl.ds(rows_per_tile * _NS, rem)])
    plsc.subcore_barrier()

    def body(g, carry):
        for b in range(_NBUF):
            j = g * _NBUF + b
            store(j - _NBUF, b).wait()
            gather(j, b).start()
            b2 = (b - _LAG) % _NBUF
            gather(j - _LAG, b2).wait()
            store(j - _LAG, b2).start()
        return carry

    lax.fori_loop(1, nfull // _NBUF, body, 0)

    for j in range(nfull - _LAG, nfull):
        gather(j, j % _NBUF).wait()
        store(j, j % _NBUF).start()
    for j in range(nfull - _NBUF, nfull):
        store(j, j % _NBUF).wait()
    for j in range(nfull, nchunks):
        gather(j, 0).start()
        gather(j, 0).wait()
        store(j, 0).start()
        store(j, 0).wait()


def kernel(x, seed_inverse_ids):
    idx = seed_inverse_ids.astype(jnp.int32)
    B = idx.shape[0]
    mesh = plsc.VectorSubcoreMesh(core_axis_name="c", subcore_axis_name="s")
    k = pl.kernel(
        _gather_body,
        mesh=mesh,
        out_type=jax.ShapeDtypeStruct((B, x.shape[1]), x.dtype),
        scratch_types=[
            pltpu.VMEM_SHARED(x.shape, x.dtype),
            pltpu.VMEM((B // _NW,), jnp.int32),
            pltpu.VMEM((_NBUF, _C, x.shape[1]), jnp.float32),
            pltpu.SemaphoreType.DMA,
            pltpu.SemaphoreType.DMA((_NBUF,)),
            pltpu.SemaphoreType.DMA((_NBUF,)),
        ],
    )
    return k(x, idx)


# speedup vs baseline: 1.5958x; 1.0022x over previous
import jax
import jax.numpy as jnp
from jax import lax
from jax.experimental import pallas as pl
from jax.experimental.pallas import tpu as pltpu
from jax.experimental.pallas import tpu_sc as plsc

_NC = 2
_NS = 16
_NW = _NC * _NS
_C = 40
_NBUF = 5
_LAG = 3


def _gather_body(x_hbm, idx_hbm, out_hbm, x_sp, idx_v, rows_v,
                 stage_sem, gsem, ssem):
    bpw = idx_hbm.shape[0] // _NW
    nchunks = bpw // _C
    nfull = nchunks // _NBUF * _NBUF
    sid = lax.axis_index("s")
    wid = sid * _NC + lax.axis_index("c")
    base0 = wid * bpw

    nrows = x_hbm.shape[0]
    rows_per_tile = (nrows // _NS) // 8 * 8
    rem = nrows - rows_per_tile * _NS
    stage = pltpu.make_async_copy(
        x_hbm.at[pl.ds(sid * rows_per_tile, rows_per_tile)],
        x_sp.at[pl.ds(sid * rows_per_tile, rows_per_tile)], stage_sem)
    stage.start()
    pltpu.sync_copy(idx_hbm.at[pl.ds(base0, bpw)], idx_v)

    def gather_hbm(j, b):
        return pltpu.make_async_copy(
            x_hbm.at[idx_v.at[pl.ds(j * _C, _C)]], rows_v.at[b], gsem.at[b])

    def gather(j, b):
        return pltpu.make_async_copy(
            x_sp.at[idx_v.at[pl.ds(j * _C, _C)]], rows_v.at[b], gsem.at[b])

    def store(j, b):
        return pltpu.make_async_copy(
            rows_v.at[b], out_hbm.at[pl.ds(base0 + j * _C, _C)], ssem.at[b])

    for t in range(_NBUF):
        gather_hbm(t, t).start()
        if t >= _LAG:
            gather_hbm(t - _LAG, t - _LAG).wait()
            store(t - _LAG, t - _LAG).start()
    stage.wait()
    if rem:
        @pl.when(sid == 0)
        def _():
            pltpu.sync_copy(x_hbm.at[pl.ds(rows_per_tile * _NS, rem)],
                            x_sp.at[pl.ds(rows_per_tile * _NS, rem)])
    plsc.subcore_barrier()

    def body(g, carry):
        for b in range(_NBUF):
            j = g * _NBUF + b
            store(j - _NBUF, b).wait()
            gather(j, b).start()
            b2 = (b - _LAG) % _NBUF
            gather(j - _LAG, b2).wait()
            store(j - _LAG, b2).start()
        return carry

    lax.fori_loop(1, nfull // _NBUF, body, 0)

    for j in range(nfull - _LAG, nfull):
        gather(j, j % _NBUF).wait()
        store(j, j % _NBUF).start()
    for j in range(nfull - _NBUF, nfull):
        store(j, j % _NBUF).wait()
    for j in range(nfull, nchunks):
        gather(j, 0).start()
        gather(j, 0).wait()
        store(j, 0).start()
        store(j, 0).wait()


def kernel(x, seed_inverse_ids):
    idx = seed_inverse_ids.astype(jnp.int32)
    B = idx.shape[0]
    mesh = plsc.VectorSubcoreMesh(core_axis_name="c", subcore_axis_name="s")
    k = pl.kernel(
        _gather_body,
        mesh=mesh,
        out_type=jax.ShapeDtypeStruct((B, x.shape[1]), x.dtype),
        scratch_types=[
            pltpu.VMEM_SHARED(x.shape, x.dtype),
            pltpu.VMEM((B // _NW,), jnp.int32),
            pltpu.VMEM((_NBUF, _C, x.shape[1]), jnp.float32),
            pltpu.SemaphoreType.DMA,
            pltpu.SemaphoreType.DMA((_NBUF,)),
            pltpu.SemaphoreType.DMA((_NBUF,)),
        ],
    )
    return k(x, idx)
